# baseline (speedup 1.0000x reference)
import functools

import jax
import jax.numpy as jnp
from jax import lax
from jax.experimental import pallas as pl
from jax.experimental.pallas import tpu as pltpu
from jax.experimental.pallas import tpu_sc as plsc

EPS = 1e-12
LANES = 16
NUM_WORKERS = 32
CHUNK = 16
SB = 4

_GATHER_DNUMS = lax.GatherDimensionNumbers(
    offset_dims=(), collapsed_slice_dims=(0,), start_index_map=(0,))


def _shuffle(v, ix):
    return lax.gather(v, ix[:, None], _GATHER_DNUMS, slice_sizes=(1,),
                      mode=lax.GatherScatterMode.PROMISE_IN_BOUNDS)


def _allsum(v, perms):
    for ix in perms:
        v = v + _shuffle(v, ix)
    return v


def _rsqrt(x):
    i = plsc.bitcast(x, jnp.int32)
    y = plsc.bitcast(jnp.int32(0x5F3759DF) - (i >> 1), jnp.float32)
    half_x = x * jnp.float32(0.5)
    for _ in range(3):
        y = y * (jnp.float32(1.5) - half_x * y * y)
    return y


def _make_sc_kernel(n_tokens, d, per_worker):
    n_chunks = per_worker // CHUNK
    n_slices = d // LANES
    blk = n_slices // SB
    mesh = plsc.VectorSubcoreMesh(core_axis_name="c", subcore_axis_name="s")

    @functools.partial(
        pl.kernel,
        mesh=mesh,
        out_type=jax.ShapeDtypeStruct((n_tokens, d), jnp.float32),
        compiler_params=pltpu.CompilerParams(needs_layout_passes=False),
        scratch_types=[
            pltpu.VMEM((per_worker,), jnp.int32),
            pltpu.VMEM((per_worker,), jnp.int32),
            pltpu.VMEM((per_worker,), jnp.int32),
            [pltpu.VMEM((CHUNK, d), jnp.float32)] * 2,
            [pltpu.VMEM((CHUNK, d // 2), jnp.int32)] * 2,
            [pltpu.VMEM((CHUNK, d), jnp.float32)] * 2,
            pltpu.VMEM((2, d), jnp.float32),
            pltpu.VMEM((CHUNK, LANES), jnp.float32),
            pltpu.VMEM((CHUNK, LANES), jnp.float32),
            pltpu.VMEM((d,), jnp.float32),
            pltpu.VMEM((d,), jnp.float32),
            pltpu.HBM((2 * 2 * 512, d // 2), jnp.int32),
            [pltpu.SemaphoreType.DMA] * 2,
            [pltpu.SemaphoreType.DMA] * 2,
        ],
    )
    def sc_kernel(tok_hbm, pos_hbm, seg_hbm, ttab_hbm, ptab_hbm, stab_hbm,
                  gamma_hbm, beta_hbm, out_hbm,
                  idx_t, idx_p, idx_c, rows_t, rows_c, rows_o, seg_v,
                  mean_v, inv_v, g_v, b_v, cps_tab, sem_g, sem_o):
        sid = lax.axis_index("s")
        cid = lax.axis_index("c")
        wid = sid * 2 + cid
        base = wid * per_worker

        pltpu.sync_copy(tok_hbm.at[wid], idx_t)
        pltpu.sync_copy(pos_hbm.at[wid], idx_p)
        pltpu.sync_copy(seg_hbm.at[wid], idx_c)
        cbase = jnp.broadcast_to(cid * 1024, (LANES,)).astype(jnp.int32)
        for i in range(per_worker // LANES):
            sl = pl.ds(i * LANES, LANES)
            idx_c[sl] = idx_p[sl] * 2 + idx_c[sl] + cbase

        pltpu.async_copy(ttab_hbm.at[idx_t.at[pl.ds(0, CHUNK)]],
                         rows_t[0], sem_g[0])

        pltpu.sync_copy(ptab_hbm.at[pl.ds(sid * 32, CHUNK)], rows_o[0])
        pltpu.sync_copy(ptab_hbm.at[pl.ds(sid * 32 + CHUNK, CHUNK)], rows_o[1])
        pltpu.sync_copy(stab_hbm, seg_v)
        for r in range(4):
            src = rows_o[r // 2]
            off = (r % 2) * 8

            def build_row(j, _, src=src, off=off):
                lp = off + (j >> 1)
                sbit = j & 1
                for i in range(n_slices // 2):
                    sl0 = pl.ds((2 * i) * LANES, LANES)
                    sl1 = pl.ds((2 * i + 1) * LANES, LANES)
                    a = src[lp, sl0] + seg_v[sbit, sl0]
                    b_ = src[lp, sl1] + seg_v[sbit, sl1]
                    packed = plsc.pack(a, b_, format=plsc.PackFormat.INTERLEAVED)
                    rows_c[0][j, pl.ds(i * LANES, LANES)] = plsc.bitcast(
                        packed, jnp.int32)
                return _

            lax.fori_loop(0, CHUNK, build_row, None)
            pltpu.sync_copy(
                rows_c[0],
                cps_tab.at[pl.ds(cid * 1024 + sid * 64 + r * CHUNK, CHUNK)])
        pltpu.sync_copy(gamma_hbm, g_v)
        pltpu.sync_copy(beta_hbm, b_v)
        plsc.subcore_barrier()

        lane = lax.iota(jnp.int32, LANES)
        perms = [lane ^ (1 << k) for k in range(4)]

        def issue_gathers(ci, b):
            isl = pl.ds(ci * CHUNK, CHUNK)
            pltpu.async_copy(ttab_hbm.at[idx_t.at[isl]], rows_t[b], sem_g[b])
            pltpu.async_copy(cps_tab.at[idx_c.at[isl]], rows_c[b], sem_g[b])

        def drain_gathers(b):
            pltpu.make_async_copy(
                ttab_hbm.at[pl.ds(0, CHUNK)], rows_t[b], sem_g[b]).wait()
            pltpu.make_async_copy(
                cps_tab.at[pl.ds(0, CHUNK)], rows_c[b], sem_g[b]).wait()

        def drain_out(b):
            pltpu.make_async_copy(
                rows_o[b], out_hbm.at[pl.ds(0, CHUNK)], sem_o[b]).wait()

        pltpu.async_copy(cps_tab.at[idx_c.at[pl.ds(0, CHUNK)]],
                         rows_c[0], sem_g[0])

        def chunk_body(ci, _):
            b = lax.rem(ci, 2)
            for bb in range(2):
                @pl.when(b == bb)
                def _(bb=bb):
                    rt, rc, ro = rows_t[bb], rows_c[bb], rows_o[bb]
                    drain_gathers(bb)

                    @pl.when(ci + 1 < n_chunks)
                    def _():
                        issue_gathers(ci + 1, 1 - bb)

                    @pl.when(ci >= 2)
                    def _():
                        drain_out(bb)

                    @plsc.parallel_loop(0, CHUNK, unroll=4)
                    def tok_body(t):
                        acc1 = jnp.zeros((LANES,), jnp.float32)
                        acc2 = jnp.zeros((LANES,), jnp.float32)
                        for i in range(n_slices // 2):
                            cv = plsc.bitcast(rc[t, pl.ds(i * LANES, LANES)],
                                              jnp.bfloat16)
                            c0, c1 = plsc.unpack(
                                cv, format=plsc.PackFormat.INTERLEAVED)
                            for h, c in ((0, c0), (1, c1)):
                                sl = pl.ds((2 * i + h) * LANES, LANES)
                                v = rt[t, sl] + c
                                ro[t, sl] = v
                                acc1 = acc1 + v
                                acc2 = acc2 + v * v
                        mean = _allsum(acc1, perms) * jnp.float32(1.0 / d)
                        ex2 = _allsum(acc2, perms) * jnp.float32(1.0 / d)
                        mean_v[t] = mean
                        inv_v[t] = _rsqrt(ex2 - mean * mean + jnp.float32(EPS))

                    for sb in range(SB):
                        gl = [g_v[pl.ds((sb * blk + i) * LANES, LANES)]
                              for i in range(blk)]
                        bl = [b_v[pl.ds((sb * blk + i) * LANES, LANES)]
                              for i in range(blk)]

                        @plsc.parallel_loop(0, CHUNK, unroll=4)
                        def norm_body(t, gl=gl, bl=bl, sb=sb):
                            mean = mean_v[t]
                            inv = inv_v[t]
                            for i in range(blk):
                                sl = pl.ds((sb * blk + i) * LANES, LANES)
                                x = (ro[t, sl] - mean) * inv
                                ro[t, sl] = x * gl[i] + bl[i]

                    pltpu.async_copy(
                        ro, out_hbm.at[pl.ds(base + ci * CHUNK, CHUNK)],
                        sem_o[bb])
            return _

        lax.fori_loop(0, n_chunks, chunk_body, None)
        for bb in range(2):
            drain_out(bb)

    return sc_kernel


def kernel(token_ids, segment_ids, pos_ids, token_table, pos_table, seg_table,
           gamma, beta):
    b, l = token_ids.shape
    _, d = token_table.shape
    n = b * l
    per_worker = n // NUM_WORKERS
    shape2 = (NUM_WORKERS, per_worker)
    tok = token_ids.reshape(shape2).astype(jnp.int32)
    pos = pos_ids.reshape(shape2).astype(jnp.int32)
    seg = segment_ids.reshape(shape2).astype(jnp.int32)
    sc = _make_sc_kernel(n, d, per_worker)
    out = sc(tok, pos, seg, token_table, pos_table, seg_table, gamma, beta)
    return out.reshape(b, l, d)


# speedup vs baseline: 1.1026x; 1.1026x over previous
import functools

import jax
import jax.numpy as jnp
from jax import lax
from jax.experimental import pallas as pl
from jax.experimental.pallas import tpu as pltpu
from jax.experimental.pallas import tpu_sc as plsc

EPS = 1e-12
LANES = 16
NUM_WORKERS = 32
CHUNK = 16
SB = 4

_GATHER_DNUMS = lax.GatherDimensionNumbers(
    offset_dims=(), collapsed_slice_dims=(0,), start_index_map=(0,))


def _shuffle(v, ix):
    return lax.gather(v, ix[:, None], _GATHER_DNUMS, slice_sizes=(1,),
                      mode=lax.GatherScatterMode.PROMISE_IN_BOUNDS)


def _allsum(v, perms):
    for ix in perms:
        v = v + _shuffle(v, ix)
    return v


def _rsqrt(x):
    i = plsc.bitcast(x, jnp.int32)
    y = plsc.bitcast(jnp.int32(0x5F3759DF) - (i >> 1), jnp.float32)
    half_x = x * jnp.float32(0.5)
    for _ in range(3):
        y = y * (jnp.float32(1.5) - half_x * y * y)
    return y


def _make_sc_kernel(n_tokens, d, per_worker):
    n_chunks = per_worker // CHUNK
    n_slices = d // LANES
    blk = n_slices // SB
    mesh = plsc.VectorSubcoreMesh(core_axis_name="c", subcore_axis_name="s")

    @functools.partial(
        pl.kernel,
        mesh=mesh,
        out_type=jax.ShapeDtypeStruct((n_tokens, d), jnp.float32),
        compiler_params=pltpu.CompilerParams(needs_layout_passes=False),
        scratch_types=[
            pltpu.VMEM((per_worker,), jnp.int32),
            pltpu.VMEM((per_worker,), jnp.int32),
            pltpu.VMEM((per_worker,), jnp.int32),
            [pltpu.VMEM((CHUNK, d), jnp.float32)] * 2,
            [pltpu.VMEM((CHUNK, d // 2), jnp.int32)] * 2,
            [pltpu.VMEM((CHUNK, d), jnp.float32)] * 2,
            pltpu.VMEM((2, d), jnp.float32),
            pltpu.VMEM((CHUNK, LANES), jnp.float32),
            pltpu.VMEM((CHUNK, LANES), jnp.float32),
            pltpu.VMEM((d,), jnp.float32),
            pltpu.VMEM((d,), jnp.float32),
            pltpu.HBM((2 * 2 * 512, d // 2), jnp.int32),
            [pltpu.SemaphoreType.DMA] * 2,
            [pltpu.SemaphoreType.DMA] * 2,
        ],
    )
    def sc_kernel(tok_hbm, pos_hbm, seg_hbm, ttab_hbm, ptab_hbm, stab_hbm,
                  gamma_hbm, beta_hbm, out_hbm,
                  idx_t, idx_p, idx_c, rows_t, rows_c, rows_o, seg_v,
                  mean_v, inv_v, g_v, b_v, cps_tab, sem_g, sem_o):
        sid = lax.axis_index("s")
        cid = lax.axis_index("c")
        wid = sid * 2 + cid
        base = wid * per_worker

        pltpu.sync_copy(tok_hbm.at[wid], idx_t)
        pltpu.sync_copy(pos_hbm.at[wid], idx_p)
        pltpu.sync_copy(seg_hbm.at[wid], idx_c)
        cbase = jnp.broadcast_to(cid * 1024, (LANES,)).astype(jnp.int32)
        for i in range(per_worker // LANES):
            sl = pl.ds(i * LANES, LANES)
            idx_c[sl] = idx_p[sl] * 2 + idx_c[sl] + cbase

        pltpu.async_copy(ttab_hbm.at[idx_t.at[pl.ds(0, CHUNK)]],
                         rows_t[0], sem_g[0])

        pltpu.sync_copy(ptab_hbm.at[pl.ds(sid * 32, CHUNK)], rows_o[0])
        pltpu.sync_copy(ptab_hbm.at[pl.ds(sid * 32 + CHUNK, CHUNK)], rows_o[1])
        pltpu.sync_copy(stab_hbm, seg_v)
        for r in range(4):
            src = rows_o[r // 2]
            off = (r % 2) * 8

            def build_row(j, _, src=src, off=off):
                lp = off + (j >> 1)
                sbit = j & 1
                for i in range(n_slices // 2):
                    sl0 = pl.ds((2 * i) * LANES, LANES)
                    sl1 = pl.ds((2 * i + 1) * LANES, LANES)
                    a = src[lp, sl0] + seg_v[sbit, sl0]
                    b_ = src[lp, sl1] + seg_v[sbit, sl1]
                    packed = plsc.pack(a, b_, format=plsc.PackFormat.INTERLEAVED)
                    rows_c[0][j, pl.ds(i * LANES, LANES)] = plsc.bitcast(
                        packed, jnp.int32)
                return _

            lax.fori_loop(0, CHUNK, build_row, None)
            pltpu.sync_copy(
                rows_c[0],
                cps_tab.at[pl.ds(cid * 1024 + sid * 64 + r * CHUNK, CHUNK)])
        pltpu.sync_copy(gamma_hbm, g_v)
        pltpu.sync_copy(beta_hbm, b_v)
        plsc.subcore_barrier()

        lane = lax.iota(jnp.int32, LANES)
        perms = [lane ^ (1 << k) for k in range(4)]

        def issue_gathers(ci, b):
            isl = pl.ds(ci * CHUNK, CHUNK)
            pltpu.async_copy(ttab_hbm.at[idx_t.at[isl]], rows_t[b], sem_g[b])
            pltpu.async_copy(cps_tab.at[idx_c.at[isl]], rows_c[b], sem_g[b])

        def drain_gathers(b):
            pltpu.make_async_copy(
                ttab_hbm.at[pl.ds(0, CHUNK)], rows_t[b], sem_g[b]).wait()
            pltpu.make_async_copy(
                cps_tab.at[pl.ds(0, CHUNK)], rows_c[b], sem_g[b]).wait()

        def drain_out(b):
            pltpu.make_async_copy(
                rows_o[b], out_hbm.at[pl.ds(0, CHUNK)], sem_o[b]).wait()

        pltpu.async_copy(cps_tab.at[idx_c.at[pl.ds(0, CHUNK)]],
                         rows_c[0], sem_g[0])

        def chunk_body(ci, _):
            b = lax.rem(ci, 2)
            for bb in range(2):
                @pl.when(b == bb)
                def _(bb=bb):
                    rt, rc, ro = rows_t[bb], rows_c[bb], rows_o[bb]
                    drain_gathers(bb)

                    @pl.when(ci + 1 < n_chunks)
                    def _():
                        issue_gathers(ci + 1, 1 - bb)

                    @pl.when(ci >= 2)
                    def _():
                        drain_out(bb)

                    @plsc.parallel_loop(0, CHUNK, unroll=2)
                    def tok_body(t):
                        acc1 = jnp.zeros((LANES,), jnp.float32)
                        acc2 = jnp.zeros((LANES,), jnp.float32)
                        for i in range(n_slices // 2):
                            cv = plsc.bitcast(rc[t, pl.ds(i * LANES, LANES)],
                                              jnp.bfloat16)
                            c0, c1 = plsc.unpack(
                                cv, format=plsc.PackFormat.INTERLEAVED)
                            for h, c in ((0, c0), (1, c1)):
                                sl = pl.ds((2 * i + h) * LANES, LANES)
                                v = rt[t, sl] + c
                                ro[t, sl] = v
                                acc1 = acc1 + v
                                acc2 = acc2 + v * v
                        mean = _allsum(acc1, perms) * jnp.float32(1.0 / d)
                        ex2 = _allsum(acc2, perms) * jnp.float32(1.0 / d)
                        mean_v[t] = mean
                        inv_v[t] = _rsqrt(ex2 - mean * mean + jnp.float32(EPS))

                    for sb in range(SB):
                        gl = [g_v[pl.ds((sb * blk + i) * LANES, LANES)]
                              for i in range(blk)]
                        bl = [b_v[pl.ds((sb * blk + i) * LANES, LANES)]
                              for i in range(blk)]

                        @plsc.parallel_loop(0, CHUNK, unroll=2)
                        def norm_body(t, gl=gl, bl=bl, sb=sb):
                            mean = mean_v[t]
                            inv = inv_v[t]
                            for i in range(blk):
                                sl = pl.ds((sb * blk + i) * LANES, LANES)
                                x = (ro[t, sl] - mean) * inv
                                ro[t, sl] = x * gl[i] + bl[i]

                    pltpu.async_copy(
                        ro, out_hbm.at[pl.ds(base + ci * CHUNK, CHUNK)],
                        sem_o[bb])
            return _

        lax.fori_loop(0, n_chunks, chunk_body, None)
        for bb in range(2):
            drain_out(bb)

    return sc_kernel


def kernel(token_ids, segment_ids, pos_ids, token_table, pos_table, seg_table,
           gamma, beta):
    b, l = token_ids.shape
    _, d = token_table.shape
    n = b * l
    per_worker = n // NUM_WORKERS
    shape2 = (NUM_WORKERS, per_worker)
    tok = token_ids.reshape(shape2).astype(jnp.int32)
    pos = pos_ids.reshape(shape2).astype(jnp.int32)
    seg = segment_ids.reshape(shape2).astype(jnp.int32)
    sc = _make_sc_kernel(n, d, per_worker)
    out = sc(tok, pos, seg, token_table, pos_table, seg_table, gamma, beta)
    return out.reshape(b, l, d)
